# baseline (speedup 1.0000x reference)
import functools
import math

import jax
import jax.numpy as jnp
import numpy as np
from jax import lax
from jax.experimental import pallas as pl
from jax.experimental.pallas import tpu as pltpu
from jax.experimental.pallas import tpu_sc as plsc

VOCAB = 100000
HIDDEN = 1024
ROT = 128
BASE_LEN = 2048
STAGE1 = 4096
MAXLEN = 8192
THETA = 10000.0
SCALE = math.sqrt(HIDDEN)

NC = 2
NS = 16
L = 16
NW = NC * NS
FHALF = ROT // 2

CHUNK = 32
NBUF = 3


def _find_correction_dim(num_rotations, dim, base, max_pos):
    return (dim * math.log(max_pos / (num_rotations * 2.0 * math.pi))) / (
        2.0 * math.log(base))


def _yarn_scale_np(inv_freq, scale, orig_len, beta_fast=32.0, beta_slow=1.0):
    dim_half = inv_freq.shape[0]
    low = max(math.floor(_find_correction_dim(beta_fast, ROT, THETA, orig_len)), 0)
    high = min(math.ceil(_find_correction_dim(beta_slow, ROT, THETA, orig_len)),
               dim_half - 1)
    ramp = np.clip(
        (np.arange(dim_half, dtype=np.float32) - low) / max(high - low, 1e-3),
        0.0, 1.0).astype(np.float32)
    extrap_mask = (1.0 - ramp).astype(np.float32)
    inv_freq_interp = (inv_freq / np.float32(scale)).astype(np.float32)
    return (inv_freq_interp * (1.0 - extrap_mask)
            + inv_freq * extrap_mask).astype(np.float32)


def _inv_freq_np(target_len):
    inv_freq = (1.0 / (np.float32(THETA) ** (
        np.arange(0, ROT, 2, dtype=np.float32) / np.float32(ROT)))).astype(
            np.float32)
    if target_len > BASE_LEN:
        inv_freq = _yarn_scale_np(inv_freq, float(STAGE1) / float(BASE_LEN),
                                  BASE_LEN)
    if target_len > STAGE1:
        inv_freq = _yarn_scale_np(inv_freq, float(MAXLEN) / float(STAGE1),
                                  STAGE1)
    return inv_freq


def _freqs_tc(invf, seq_len):
    blk = 512

    def body(invf_ref, out_ref):
        pid = pl.program_id(0)
        t = (pid * blk + lax.broadcasted_iota(jnp.int32, (blk, FHALF), 0)
             ).astype(jnp.float32)
        out_ref[...] = t * invf_ref[...]

    return pl.pallas_call(
        body,
        grid=(seq_len // blk,),
        in_specs=[pl.BlockSpec((1, FHALF), lambda i: (0, 0))],
        out_specs=pl.BlockSpec((blk, FHALF), lambda i: (i, 0)),
        out_shape=jax.ShapeDtypeStruct((seq_len, FHALF), jnp.float32),
    )(invf.reshape(1, FHALF))


def _make_sc_call(batch, seq_len):
    n_tok = batch * seq_len
    assert n_tok % NW == 0
    tok_per_w = n_tok // NW
    spans_per_row = seq_len // tok_per_w
    assert tok_per_w % CHUNK == 0
    n_chunks = tok_per_w // CHUNK

    mesh = plsc.VectorSubcoreMesh(core_axis_name="c", subcore_axis_name="s")

    @functools.partial(
        pl.kernel,
        mesh=mesh,
        out_type=jax.ShapeDtypeStruct((batch, seq_len, HIDDEN), jnp.float32),
        scratch_types=[
            pltpu.VMEM((tok_per_w,), jnp.int32),
            pltpu.VMEM((CHUNK, HIDDEN), jnp.float32),
            pltpu.VMEM((CHUNK, HIDDEN), jnp.float32),
            pltpu.VMEM((CHUNK, HIDDEN), jnp.float32),
            pltpu.SemaphoreType.DMA,
            pltpu.SemaphoreType.DMA,
            pltpu.SemaphoreType.DMA,
            pltpu.SemaphoreType.DMA,
            pltpu.SemaphoreType.DMA,
            pltpu.SemaphoreType.DMA,
        ],
    )
    def sc_call(ids_hbm, table_hbm, x_hbm,
                idx_all, rows0, rows1, rows2,
                gsem0, gsem1, gsem2, ssem0, ssem1, ssem2):
        wid = lax.axis_index("s") * NC + lax.axis_index("c")
        rows = (rows0, rows1, rows2)
        gsem = (gsem0, gsem1, gsem2)
        ssem = (ssem0, ssem1, ssem2)
        bidx = wid // spans_per_row
        soff = (wid % spans_per_row) * tok_per_w

        pltpu.sync_copy(ids_hbm.at[bidx, pl.ds(soff, tok_per_w)], idx_all)

        def gather(g, b):
            return pltpu.async_copy(
                table_hbm.at[idx_all.at[pl.ds(g * CHUNK, CHUNK)]],
                rows[b], gsem[b])

        def wait_gather(g, b):
            pltpu.make_async_copy(
                table_hbm.at[idx_all.at[pl.ds(g * CHUNK, CHUNK)]],
                rows[b], gsem[b]).wait()

        def scatter(g, b):
            return pltpu.async_copy(
                rows[b],
                x_hbm.at[bidx, pl.ds(soff + g * CHUNK, CHUNK)],
                ssem[b])

        def wait_scatter(g, b):
            pltpu.make_async_copy(
                rows[b],
                x_hbm.at[bidx, pl.ds(soff + g * CHUNK, CHUNK)],
                ssem[b]).wait()

        def scale_rows(rv):
            def row(r, c2):
                def vec(j, c3):
                    sl = pl.ds(j * L, L)
                    rv[r, sl] = rv[r, sl] * SCALE
                    return c3
                lax.fori_loop(0, HIDDEN // L, vec, None, unroll=4)
                return c2
            lax.fori_loop(0, CHUNK, row, None)

        gather(0, 0)
        gather(1, 1)

        wait_gather(0, 0)
        scale_rows(rows[0])
        scatter(0, 0)
        gather(2, 2)
        wait_gather(1, 1)
        scale_rows(rows[1])
        scatter(1, 1)
        wait_scatter(0, 0)
        gather(3, 0)

        def block(g0):
            for b in range(NBUF):
                g = g0 + b
                bb = (2 + b) % NBUF
                nb = (2 + b + 2) % NBUF
                wait_gather(g, bb)
                scale_rows(rows[bb])
                scatter(g, bb)
                wait_scatter(g - 1, nb)
                gather(g + 2, nb)

        assert (n_chunks - 2 - 2) % NBUF == 0
        pl.loop(2, n_chunks - 4, step=NBUF)(block)

        for g in (n_chunks - 2, n_chunks - 1):
            b = g % NBUF
            wait_gather(g, b)
            scale_rows(rows[b])
            scatter(g, b)
        for g in range(n_chunks - NBUF, n_chunks):
            wait_scatter(g, g % NBUF)

    return sc_call


def kernel(input_ids, token_embed_weight):
    batch, seq_len = input_ids.shape
    invf = jnp.asarray(_inv_freq_np(seq_len))
    sc_call = _make_sc_call(batch, seq_len)
    x = sc_call(input_ids, token_embed_weight)
    freqs = _freqs_tc(invf, seq_len)
    return x, freqs


# speedup vs baseline: 2.7990x; 2.7990x over previous
import functools
import math

import jax
import jax.numpy as jnp
import numpy as np
from jax import lax
from jax.experimental import pallas as pl
from jax.experimental.pallas import tpu as pltpu
from jax.experimental.pallas import tpu_sc as plsc

VOCAB = 100000
HIDDEN = 1024
ROT = 128
BASE_LEN = 2048
STAGE1 = 4096
MAXLEN = 8192
THETA = 10000.0
SCALE = math.sqrt(HIDDEN)

NC = 2
NS = 16
L = 16
NW = NC * NS
FHALF = ROT // 2

CHUNK = 32
NBUF = 3


def _find_correction_dim(num_rotations, dim, base, max_pos):
    return (dim * math.log(max_pos / (num_rotations * 2.0 * math.pi))) / (
        2.0 * math.log(base))


def _yarn_scale_np(inv_freq, scale, orig_len, beta_fast=32.0, beta_slow=1.0):
    dim_half = inv_freq.shape[0]
    low = max(math.floor(_find_correction_dim(beta_fast, ROT, THETA, orig_len)), 0)
    high = min(math.ceil(_find_correction_dim(beta_slow, ROT, THETA, orig_len)),
               dim_half - 1)
    ramp = np.clip(
        (np.arange(dim_half, dtype=np.float32) - low) / max(high - low, 1e-3),
        0.0, 1.0).astype(np.float32)
    extrap_mask = (1.0 - ramp).astype(np.float32)
    inv_freq_interp = (inv_freq / np.float32(scale)).astype(np.float32)
    return (inv_freq_interp * (1.0 - extrap_mask)
            + inv_freq * extrap_mask).astype(np.float32)


def _inv_freq_np(target_len):
    inv_freq = (1.0 / (np.float32(THETA) ** (
        np.arange(0, ROT, 2, dtype=np.float32) / np.float32(ROT)))).astype(
            np.float32)
    if target_len > BASE_LEN:
        inv_freq = _yarn_scale_np(inv_freq, float(STAGE1) / float(BASE_LEN),
                                  BASE_LEN)
    if target_len > STAGE1:
        inv_freq = _yarn_scale_np(inv_freq, float(MAXLEN) / float(STAGE1),
                                  STAGE1)
    return inv_freq


def _freqs_tc(invf, seq_len):
    blk = 512

    def body(invf_ref, out_ref):
        pid = pl.program_id(0)
        t = (pid * blk + lax.broadcasted_iota(jnp.int32, (blk, FHALF), 0)
             ).astype(jnp.float32)
        out_ref[...] = t * invf_ref[...]

    return pl.pallas_call(
        body,
        grid=(seq_len // blk,),
        in_specs=[pl.BlockSpec((1, FHALF), lambda i: (0, 0))],
        out_specs=pl.BlockSpec((blk, FHALF), lambda i: (i, 0)),
        out_shape=jax.ShapeDtypeStruct((seq_len, FHALF), jnp.float32),
    )(invf.reshape(1, FHALF))


def _make_sc_call(batch, seq_len):
    n_tok = batch * seq_len
    assert n_tok % NW == 0
    tok_per_w = n_tok // NW
    spans_per_row = seq_len // tok_per_w
    assert tok_per_w % CHUNK == 0
    n_chunks = tok_per_w // CHUNK

    mesh = plsc.VectorSubcoreMesh(core_axis_name="c", subcore_axis_name="s")

    @functools.partial(
        pl.kernel,
        mesh=mesh,
        out_type=jax.ShapeDtypeStruct((batch, seq_len, HIDDEN), jnp.float32),
        scratch_types=[
            pltpu.VMEM((tok_per_w,), jnp.int32),
            pltpu.VMEM((CHUNK, HIDDEN), jnp.float32),
            pltpu.VMEM((CHUNK, HIDDEN), jnp.float32),
            pltpu.VMEM((CHUNK, HIDDEN), jnp.float32),
            pltpu.SemaphoreType.DMA,
            pltpu.SemaphoreType.DMA,
            pltpu.SemaphoreType.DMA,
            pltpu.SemaphoreType.DMA,
            pltpu.SemaphoreType.DMA,
            pltpu.SemaphoreType.DMA,
        ],
    )
    def sc_call(ids_hbm, table_hbm, x_hbm,
                idx_all, rows0, rows1, rows2,
                gsem0, gsem1, gsem2, ssem0, ssem1, ssem2):
        wid = lax.axis_index("s") * NC + lax.axis_index("c")
        rows = (rows0, rows1, rows2)
        gsem = (gsem0, gsem1, gsem2)
        ssem = (ssem0, ssem1, ssem2)
        bidx = wid // spans_per_row
        soff = (wid % spans_per_row) * tok_per_w

        pltpu.sync_copy(ids_hbm.at[bidx, pl.ds(soff, tok_per_w)], idx_all)

        def gather(g, b):
            return pltpu.async_copy(
                table_hbm.at[idx_all.at[pl.ds(g * CHUNK, CHUNK)]],
                rows[b], gsem[b])

        def wait_gather(g, b):
            pltpu.make_async_copy(
                table_hbm.at[idx_all.at[pl.ds(g * CHUNK, CHUNK)]],
                rows[b], gsem[b]).wait()

        def scatter(g, b):
            return pltpu.async_copy(
                rows[b],
                x_hbm.at[bidx, pl.ds(soff + g * CHUNK, CHUNK)],
                ssem[b])

        def wait_scatter(g, b):
            pltpu.make_async_copy(
                rows[b],
                x_hbm.at[bidx, pl.ds(soff + g * CHUNK, CHUNK)],
                ssem[b]).wait()

        def scale_rows(rv):
            def row(r, c2):
                def vec(j, c3):
                    sl = pl.ds(j * L, L)
                    rv[r, sl] = rv[r, sl] * SCALE
                    return c3
                lax.fori_loop(0, HIDDEN // L, vec, None, unroll=8)
                return c2
            lax.fori_loop(0, CHUNK, row, None)

        gather(0, 0)
        gather(1, 1)

        wait_gather(0, 0)
        scale_rows(rows[0])
        scatter(0, 0)
        gather(2, 2)
        wait_gather(1, 1)
        scale_rows(rows[1])
        scatter(1, 1)
        wait_scatter(0, 0)
        gather(3, 0)

        def block(g0):
            for b in range(NBUF):
                g = g0 + b
                bb = (2 + b) % NBUF
                nb = (2 + b + 2) % NBUF
                wait_gather(g, bb)
                scale_rows(rows[bb])
                scatter(g, bb)
                wait_scatter(g - 1, nb)
                gather(g + 2, nb)

        assert (n_chunks - 2 - 2) % NBUF == 0
        pl.loop(2, n_chunks - 4, step=NBUF)(block)

        for g in (n_chunks - 2, n_chunks - 1):
            b = g % NBUF
            wait_gather(g, b)
            scale_rows(rows[b])
            scatter(g, b)
        for g in range(n_chunks - NBUF, n_chunks):
            wait_scatter(g, g % NBUF)

    return sc_call


def kernel(input_ids, token_embed_weight):
    batch, seq_len = input_ids.shape
    invf = jnp.asarray(_inv_freq_np(seq_len))
    sc_call = _make_sc_call(batch, seq_len)
    x = sc_call(input_ids, token_embed_weight)
    freqs = _freqs_tc(invf, seq_len)
    return x, freqs


# speedup vs baseline: 2.7990x; 1.0000x over previous
import functools
import math

import jax
import jax.numpy as jnp
import numpy as np
from jax import lax
from jax.experimental import pallas as pl
from jax.experimental.pallas import tpu as pltpu
from jax.experimental.pallas import tpu_sc as plsc

VOCAB = 100000
HIDDEN = 1024
ROT = 128
BASE_LEN = 2048
STAGE1 = 4096
MAXLEN = 8192
THETA = 10000.0
SCALE = math.sqrt(HIDDEN)

NC = 2
NS = 16
L = 16
NW = NC * NS
FHALF = ROT // 2

CHUNK = 32
NBUF = 3


def _find_correction_dim(num_rotations, dim, base, max_pos):
    return (dim * math.log(max_pos / (num_rotations * 2.0 * math.pi))) / (
        2.0 * math.log(base))


def _yarn_scale_np(inv_freq, scale, orig_len, beta_fast=32.0, beta_slow=1.0):
    dim_half = inv_freq.shape[0]
    low = max(math.floor(_find_correction_dim(beta_fast, ROT, THETA, orig_len)), 0)
    high = min(math.ceil(_find_correction_dim(beta_slow, ROT, THETA, orig_len)),
               dim_half - 1)
    ramp = np.clip(
        (np.arange(dim_half, dtype=np.float32) - low) / max(high - low, 1e-3),
        0.0, 1.0).astype(np.float32)
    extrap_mask = (1.0 - ramp).astype(np.float32)
    inv_freq_interp = (inv_freq / np.float32(scale)).astype(np.float32)
    return (inv_freq_interp * (1.0 - extrap_mask)
            + inv_freq * extrap_mask).astype(np.float32)


def _inv_freq_np(target_len):
    inv_freq = (1.0 / (np.float32(THETA) ** (
        np.arange(0, ROT, 2, dtype=np.float32) / np.float32(ROT)))).astype(
            np.float32)
    if target_len > BASE_LEN:
        inv_freq = _yarn_scale_np(inv_freq, float(STAGE1) / float(BASE_LEN),
                                  BASE_LEN)
    if target_len > STAGE1:
        inv_freq = _yarn_scale_np(inv_freq, float(MAXLEN) / float(STAGE1),
                                  STAGE1)
    return inv_freq


def _freqs_tc(invf, seq_len):
    blk = 512

    def body(invf_ref, out_ref):
        pid = pl.program_id(0)
        t = (pid * blk + lax.broadcasted_iota(jnp.int32, (blk, FHALF), 0)
             ).astype(jnp.float32)
        out_ref[...] = t * invf_ref[...]

    return pl.pallas_call(
        body,
        grid=(seq_len // blk,),
        in_specs=[pl.BlockSpec((1, FHALF), lambda i: (0, 0))],
        out_specs=pl.BlockSpec((blk, FHALF), lambda i: (i, 0)),
        out_shape=jax.ShapeDtypeStruct((seq_len, FHALF), jnp.float32),
    )(invf.reshape(1, FHALF))


def _make_sc_call(batch, seq_len):
    n_tok = batch * seq_len
    assert n_tok % NW == 0
    tok_per_w = n_tok // NW
    spans_per_row = seq_len // tok_per_w
    assert tok_per_w % CHUNK == 0
    n_chunks = tok_per_w // CHUNK

    mesh = plsc.VectorSubcoreMesh(core_axis_name="c", subcore_axis_name="s")

    @functools.partial(
        pl.kernel,
        mesh=mesh,
        out_type=jax.ShapeDtypeStruct((batch, seq_len, HIDDEN), jnp.float32),
        scratch_types=[
            pltpu.VMEM((tok_per_w,), jnp.int32),
            pltpu.VMEM((CHUNK, HIDDEN), jnp.float32),
            pltpu.VMEM((CHUNK, HIDDEN), jnp.float32),
            pltpu.VMEM((CHUNK, HIDDEN), jnp.float32),
            pltpu.SemaphoreType.DMA,
            pltpu.SemaphoreType.DMA,
            pltpu.SemaphoreType.DMA,
            pltpu.SemaphoreType.DMA,
            pltpu.SemaphoreType.DMA,
            pltpu.SemaphoreType.DMA,
        ],
    )
    def sc_call(ids_hbm, table_hbm, x_hbm,
                idx_all, rows0, rows1, rows2,
                gsem0, gsem1, gsem2, ssem0, ssem1, ssem2):
        wid = lax.axis_index("s") * NC + lax.axis_index("c")
        rows = (rows0, rows1, rows2)
        gsem = (gsem0, gsem1, gsem2)
        ssem = (ssem0, ssem1, ssem2)
        bidx = wid // spans_per_row
        soff = (wid % spans_per_row) * tok_per_w

        pltpu.sync_copy(ids_hbm.at[bidx, pl.ds(soff, tok_per_w)], idx_all)

        def gather(g, b):
            return pltpu.async_copy(
                table_hbm.at[idx_all.at[pl.ds(g * CHUNK, CHUNK)]],
                rows[b], gsem[b])

        def wait_gather(g, b):
            pltpu.make_async_copy(
                table_hbm.at[idx_all.at[pl.ds(g * CHUNK, CHUNK)]],
                rows[b], gsem[b]).wait()

        def scatter(g, b):
            return pltpu.async_copy(
                rows[b],
                x_hbm.at[bidx, pl.ds(soff + g * CHUNK, CHUNK)],
                ssem[b])

        def wait_scatter(g, b):
            pltpu.make_async_copy(
                rows[b],
                x_hbm.at[bidx, pl.ds(soff + g * CHUNK, CHUNK)],
                ssem[b]).wait()

        def scale_rows(rv):
            def row(r, c2):
                def vec(j, c3):
                    sl = pl.ds(j * L, L)
                    rv[r, sl] = rv[r, sl] * SCALE
                    return c3
                lax.fori_loop(0, HIDDEN // L, vec, None, unroll=8)
                return c2
            lax.fori_loop(0, CHUNK, row, None)

        gather(0, 0)
        gather(1, 1)

        wait_gather(0, 0)
        scale_rows(rows[0])
        scatter(0, 0)
        gather(2, 2)
        wait_gather(1, 1)
        scale_rows(rows[1])
        scatter(1, 1)
        wait_scatter(0, 0)
        gather(3, 0)

        def block(g0):
            for b in range(NBUF):
                g = g0 + b
                bb = (2 + b) % NBUF
                nb = (2 + b + 2) % NBUF
                wait_gather(g, bb)
                wait_scatter(g - 1, nb)
                gather(g + 2, nb)
                scale_rows(rows[bb])
                scatter(g, bb)

        assert (n_chunks - 2 - 2) % NBUF == 0
        pl.loop(2, n_chunks - 4, step=NBUF)(block)

        for g in (n_chunks - 2, n_chunks - 1):
            b = g % NBUF
            wait_gather(g, b)
            scale_rows(rows[b])
            scatter(g, b)
        for g in range(n_chunks - NBUF, n_chunks):
            wait_scatter(g, g % NBUF)

    return sc_call


def kernel(input_ids, token_embed_weight):
    batch, seq_len = input_ids.shape
    invf = jnp.asarray(_inv_freq_np(seq_len))
    sc_call = _make_sc_call(batch, seq_len)
    x = sc_call(input_ids, token_embed_weight)
    freqs = _freqs_tc(invf, seq_len)
    return x, freqs
